# baseline (speedup 1.0000x reference)
import jax
import jax.numpy as jnp
from jax.experimental import pallas as pl
from jax.experimental.pallas import tpu as pltpu

_N = 100000
_TILE = 2000


def _body(x_ref, wz1_ref, bz1_ref, wh1_ref, bh1_ref,
          wz2_ref, bz2_ref, wh2_ref, bh2_ref,
          wlin_ref, blin_ref, wgrad_ref, bgrad_ref, o_ref):
    x = x_ref[...]
    f32 = jnp.float32
    z1 = jax.nn.sigmoid(jnp.dot(x, wz1_ref[...], preferred_element_type=f32)
                        + bz1_ref[...])
    t1 = jnp.tanh(jnp.dot(x, wh1_ref[...], preferred_element_type=f32)
                  + bh1_ref[...])
    h1 = jax.nn.relu((1.0 - z1) * t1)
    z2 = jax.nn.sigmoid(jnp.dot(h1, wz2_ref[...], preferred_element_type=f32)
                        + bz2_ref[...])
    t2 = jnp.tanh(jnp.dot(h1, wh2_ref[...], preferred_element_type=f32)
                  + bh2_ref[...])
    h2 = jnp.tanh((1.0 - z2) * t2)
    v = jnp.dot(h2, wlin_ref[...], preferred_element_type=f32) + blin_ref[...]
    wg = wgrad_ref[...]
    bg = bgrad_ref[...]
    wg_r = jnp.concatenate([wg[:, 1:], -wg[:, :1]], axis=1)
    bg_r = jnp.concatenate([bg[:, 1:], -bg[:, :1]], axis=1)
    o_ref[...] = v * wg_r + bg_r


def kernel(x, edge_index, W_xz1, b_xz1, W_hz1, b_hz1, W_xr1, b_xr1, W_hr1,
           b_hr1, W_xh1, b_xh1, W_hh1, b_hh1, W_xz2, b_xz2, W_hz2, b_hz2,
           W_xr2, b_xr2, W_hr2, b_hr2, W_xh2, b_xh2, W_hh2, b_hh2,
           W_lin, b_lin, W_grad, b_grad):
    del edge_index
    del W_hz1, W_xr1, b_xr1, W_hr1, b_hr1, W_hh1
    del W_hz2, W_xr2, b_xr2, W_hr2, b_hr2, W_hh2
    bz1 = (b_xz1 + b_hz1).reshape(1, 32)
    bh1 = (b_xh1 + b_hh1).reshape(1, 32)
    bz2 = (b_xz2 + b_hz2).reshape(1, 128)
    bh2 = (b_xh2 + b_hh2).reshape(1, 128)
    blin = b_lin.reshape(1, 1)
    bgrad = b_grad.reshape(1, 2)

    grid = _N // _TILE
    row_spec = pl.BlockSpec((_TILE, 2), lambda i: (i, 0))

    def w_spec(a, b):
        return pl.BlockSpec((a, b), lambda i: (0, 0))

    return pl.pallas_call(
        _body,
        grid=(grid,),
        in_specs=[
            row_spec,
            w_spec(2, 32), w_spec(1, 32), w_spec(2, 32), w_spec(1, 32),
            w_spec(32, 128), w_spec(1, 128), w_spec(32, 128), w_spec(1, 128),
            w_spec(128, 1), w_spec(1, 1), w_spec(1, 2), w_spec(1, 2),
        ],
        out_specs=row_spec,
        out_shape=jax.ShapeDtypeStruct((_N, 2), jnp.float32),
        compiler_params=pltpu.CompilerParams(
            dimension_semantics=("arbitrary",),
        ),
    )(x, W_xz1, bz1, W_xh1, bh1, W_xz2, bz2, W_xh2, bh2,
      W_lin, blin, W_grad, bgrad)


# speedup vs baseline: 1.0945x; 1.0945x over previous
import jax
import jax.numpy as jnp
from jax.experimental import pallas as pl
from jax.experimental.pallas import tpu as pltpu

_N = 100000
_TILE = 4000


def _body(x_ref, wz1_ref, bz1_ref, wh1_ref, bh1_ref,
          w2_ref, b2_ref, wf_ref, bf_ref, o_ref):
    x = x_ref[...]
    f32 = jnp.float32
    z1 = jax.nn.sigmoid(jnp.dot(x, wz1_ref[...], preferred_element_type=f32)
                        + bz1_ref[...])
    t1 = jnp.tanh(jnp.dot(x, wh1_ref[...], preferred_element_type=f32)
                  + bh1_ref[...])
    h1 = jax.nn.relu((1.0 - z1) * t1)
    a2 = jnp.dot(h1, w2_ref[...], preferred_element_type=f32) + b2_ref[...]
    z2 = jax.nn.sigmoid(a2[:, :128])
    t2 = jnp.tanh(a2[:, 128:])
    h2 = jnp.tanh((1.0 - z2) * t2)
    o_ref[...] = (jnp.dot(h2, wf_ref[...], preferred_element_type=f32)
                  + bf_ref[...])


def kernel(x, edge_index, W_xz1, b_xz1, W_hz1, b_hz1, W_xr1, b_xr1, W_hr1,
           b_hr1, W_xh1, b_xh1, W_hh1, b_hh1, W_xz2, b_xz2, W_hz2, b_hz2,
           W_xr2, b_xr2, W_hr2, b_hr2, W_xh2, b_xh2, W_hh2, b_hh2,
           W_lin, b_lin, W_grad, b_grad):
    del edge_index
    del W_hz1, W_xr1, b_xr1, W_hr1, b_hr1, W_hh1
    del W_hz2, W_xr2, b_xr2, W_hr2, b_hr2, W_hh2
    bz1 = (b_xz1 + b_hz1).reshape(1, 32)
    bh1 = (b_xh1 + b_hh1).reshape(1, 32)
    w2 = jnp.concatenate([W_xz2, W_xh2], axis=1)
    b2 = jnp.concatenate([b_xz2 + b_hz2, b_xh2 + b_hh2]).reshape(1, 256)
    wf = W_lin @ W_grad
    bf = b_lin[0] * W_grad[0] + b_grad
    wf = jnp.concatenate([wf[:, 1:], -wf[:, :1]], axis=1)
    bf = jnp.stack([bf[1], -bf[0]]).reshape(1, 2)

    grid = _N // _TILE
    row_spec = pl.BlockSpec((_TILE, 2), lambda i: (i, 0))

    def w_spec(a, b):
        return pl.BlockSpec((a, b), lambda i: (0, 0))

    return pl.pallas_call(
        _body,
        grid=(grid,),
        in_specs=[
            row_spec,
            w_spec(2, 32), w_spec(1, 32), w_spec(2, 32), w_spec(1, 32),
            w_spec(32, 256), w_spec(1, 256), w_spec(128, 2), w_spec(1, 2),
        ],
        out_specs=row_spec,
        out_shape=jax.ShapeDtypeStruct((_N, 2), jnp.float32),
        compiler_params=pltpu.CompilerParams(
            dimension_semantics=("parallel",),
        ),
    )(x, W_xz1, bz1, W_xh1, bh1, w2, b2, wf, bf)


# speedup vs baseline: 1.2414x; 1.1342x over previous
import jax
import jax.numpy as jnp
from jax.experimental import pallas as pl
from jax.experimental.pallas import tpu as pltpu

_N = 100000
_TILE = 10000


def _body(x_ref, w1_ref, b1_ref, w2_ref, b2_ref, wf_ref, bf_ref, o_ref):
    x = x_ref[...]
    f32 = jnp.float32
    u1 = jnp.tanh(jnp.dot(x, w1_ref[...], preferred_element_type=f32)
                  + b1_ref[...])
    h1 = jax.nn.relu((1.0 - u1[:, :32]) * u1[:, 32:])
    u2 = jnp.tanh(jnp.dot(h1, w2_ref[...], preferred_element_type=f32)
                  + b2_ref[...])
    h2 = jnp.tanh((0.5 - 0.5 * u2[:, :128]) * u2[:, 128:])
    o_ref[...] = (jnp.dot(h2, wf_ref[...], preferred_element_type=f32)
                  + bf_ref[...])


def kernel(x, edge_index, W_xz1, b_xz1, W_hz1, b_hz1, W_xr1, b_xr1, W_hr1,
           b_hr1, W_xh1, b_xh1, W_hh1, b_hh1, W_xz2, b_xz2, W_hz2, b_hz2,
           W_xr2, b_xr2, W_hr2, b_hr2, W_xh2, b_xh2, W_hh2, b_hh2,
           W_lin, b_lin, W_grad, b_grad):
    del edge_index
    del W_hz1, W_xr1, b_xr1, W_hr1, b_hr1, W_hh1
    del W_hz2, W_xr2, b_xr2, W_hr2, b_hr2, W_hh2
    w1 = jnp.concatenate([0.5 * W_xz1, W_xh1], axis=1)
    b1 = jnp.concatenate([0.5 * (b_xz1 + b_hz1),
                          b_xh1 + b_hh1]).reshape(1, 64)
    w2 = jnp.concatenate([0.25 * W_xz2, 0.5 * W_xh2], axis=1)
    b2 = jnp.concatenate([0.5 * (b_xz2 + b_hz2),
                          b_xh2 + b_hh2]).reshape(1, 256)
    wf = W_lin @ W_grad
    bf = b_lin[0] * W_grad[0] + b_grad
    wf = jnp.concatenate([wf[:, 1:], -wf[:, :1]], axis=1)
    bf = jnp.stack([bf[1], -bf[0]]).reshape(1, 2)

    grid = _N // _TILE
    row_spec = pl.BlockSpec((_TILE, 2), lambda i: (i, 0))

    def w_spec(a, b):
        return pl.BlockSpec((a, b), lambda i: (0, 0))

    return pl.pallas_call(
        _body,
        grid=(grid,),
        in_specs=[
            row_spec,
            w_spec(2, 64), w_spec(1, 64), w_spec(32, 256), w_spec(1, 256),
            w_spec(128, 2), w_spec(1, 2),
        ],
        out_specs=row_spec,
        out_shape=jax.ShapeDtypeStruct((_N, 2), jnp.float32),
        compiler_params=pltpu.CompilerParams(
            dimension_semantics=("parallel",),
        ),
    )(x, w1, b1, w2, b2, wf, bf)


# speedup vs baseline: 3.0107x; 2.4252x over previous
import jax
import jax.numpy as jnp
from jax.experimental import pallas as pl
from jax.experimental.pallas import tpu as pltpu

_N = 100000
_TILE = 8192


def _body(x_ref, w1_ref, b1_ref, w2_ref, b2_ref, wf_ref, bf_ref, o_ref):
    x = x_ref[...]
    f32 = jnp.float32
    u1 = jnp.tanh(jnp.dot(w1_ref[...], x, preferred_element_type=f32)
                  + b1_ref[...])
    h1 = jax.nn.relu((1.0 - u1[:32, :]) * u1[32:, :])
    u2 = jnp.tanh(jnp.dot(w2_ref[...], h1, preferred_element_type=f32)
                  + b2_ref[...])
    h2 = jnp.tanh((0.5 - 0.5 * u2[:128, :]) * u2[128:, :])
    o_ref[...] = (jnp.dot(wf_ref[...], h2, preferred_element_type=f32)
                  + bf_ref[...])


def kernel(x, edge_index, W_xz1, b_xz1, W_hz1, b_hz1, W_xr1, b_xr1, W_hr1,
           b_hr1, W_xh1, b_xh1, W_hh1, b_hh1, W_xz2, b_xz2, W_hz2, b_hz2,
           W_xr2, b_xr2, W_hr2, b_hr2, W_xh2, b_xh2, W_hh2, b_hh2,
           W_lin, b_lin, W_grad, b_grad):
    del edge_index
    del W_hz1, W_xr1, b_xr1, W_hr1, b_hr1, W_hh1
    del W_hz2, W_xr2, b_xr2, W_hr2, b_hr2, W_hh2
    w1 = jnp.concatenate([0.5 * W_xz1.T, W_xh1.T], axis=0)
    b1 = jnp.concatenate([0.5 * (b_xz1 + b_hz1),
                          b_xh1 + b_hh1]).reshape(64, 1)
    w2 = jnp.concatenate([0.25 * W_xz2.T, 0.5 * W_xh2.T], axis=0)
    b2 = jnp.concatenate([0.5 * (b_xz2 + b_hz2),
                          b_xh2 + b_hh2]).reshape(256, 1)
    wf = W_lin @ W_grad
    bf = b_lin[0] * W_grad[0] + b_grad
    wf = jnp.concatenate([wf[:, 1:], -wf[:, :1]], axis=1).T
    bf = jnp.stack([bf[1], -bf[0]]).reshape(2, 1)

    xt = x.T
    grid = -(-_N // _TILE)
    col_spec = pl.BlockSpec((2, _TILE), lambda i: (0, i))

    def w_spec(a, b):
        return pl.BlockSpec((a, b), lambda i: (0, 0))

    out_t = pl.pallas_call(
        _body,
        grid=(grid,),
        in_specs=[
            col_spec,
            w_spec(64, 2), w_spec(64, 1), w_spec(256, 32), w_spec(256, 1),
            w_spec(2, 128), w_spec(2, 1),
        ],
        out_specs=col_spec,
        out_shape=jax.ShapeDtypeStruct((2, _N), jnp.float32),
        compiler_params=pltpu.CompilerParams(
            dimension_semantics=("parallel",),
        ),
    )(xt, w1, b1, w2, b2, wf, bf)
    return out_t.T


# speedup vs baseline: 3.2964x; 1.0949x over previous
import jax
import jax.numpy as jnp
from jax.experimental import pallas as pl
from jax.experimental.pallas import tpu as pltpu

_N = 100000
_TILE = 12544


def _body(x_ref, w1_ref, b1_ref, w2_ref, b2_ref, wf_ref, bf_ref, o_ref):
    x = x_ref[...]
    f32 = jnp.float32
    u1 = jnp.tanh(jnp.dot(w1_ref[...], x, preferred_element_type=f32)
                  + b1_ref[...])
    h1 = jax.nn.relu((1.0 - u1[:32, :]) * u1[32:, :])
    u2 = jnp.tanh(jnp.dot(w2_ref[...], h1, preferred_element_type=f32)
                  + b2_ref[...])
    h2 = jnp.tanh((0.5 - 0.5 * u2[:128, :]) * u2[128:, :])
    o_ref[...] = (jnp.dot(wf_ref[...], h2, preferred_element_type=f32)
                  + bf_ref[...])


def kernel(x, edge_index, W_xz1, b_xz1, W_hz1, b_hz1, W_xr1, b_xr1, W_hr1,
           b_hr1, W_xh1, b_xh1, W_hh1, b_hh1, W_xz2, b_xz2, W_hz2, b_hz2,
           W_xr2, b_xr2, W_hr2, b_hr2, W_xh2, b_xh2, W_hh2, b_hh2,
           W_lin, b_lin, W_grad, b_grad):
    del edge_index
    del W_hz1, W_xr1, b_xr1, W_hr1, b_hr1, W_hh1
    del W_hz2, W_xr2, b_xr2, W_hr2, b_hr2, W_hh2
    w1 = jnp.concatenate([0.5 * W_xz1.T, W_xh1.T], axis=0)
    b1 = jnp.concatenate([0.5 * (b_xz1 + b_hz1),
                          b_xh1 + b_hh1]).reshape(64, 1)
    w2 = jnp.concatenate([0.25 * W_xz2.T, 0.5 * W_xh2.T], axis=0)
    b2 = jnp.concatenate([0.5 * (b_xz2 + b_hz2),
                          b_xh2 + b_hh2]).reshape(256, 1)
    wf = W_lin @ W_grad
    bf = b_lin[0] * W_grad[0] + b_grad
    wf = jnp.concatenate([wf[:, 1:], -wf[:, :1]], axis=1).T
    bf = jnp.stack([bf[1], -bf[0]]).reshape(2, 1)

    xt = x.T
    grid = -(-_N // _TILE)
    col_spec = pl.BlockSpec((2, _TILE), lambda i: (0, i))

    def w_spec(a, b):
        return pl.BlockSpec((a, b), lambda i: (0, 0))

    out_t = pl.pallas_call(
        _body,
        grid=(grid,),
        in_specs=[
            col_spec,
            w_spec(64, 2), w_spec(64, 1), w_spec(256, 32), w_spec(256, 1),
            w_spec(2, 128), w_spec(2, 1),
        ],
        out_specs=col_spec,
        out_shape=jax.ShapeDtypeStruct((2, _N), jnp.float32),
        compiler_params=pltpu.CompilerParams(
            dimension_semantics=("parallel",),
        ),
    )(xt, w1, b1, w2, b2, wf, bf)
    return out_t.T


# speedup vs baseline: 3.3954x; 1.0300x over previous
import jax
import jax.numpy as jnp
from jax.experimental import pallas as pl
from jax.experimental.pallas import tpu as pltpu

_N = 100000
_TILE = 25088


def _body(x_ref, w1_ref, b1_ref, w2_ref, b2_ref, wf_ref, bf_ref, o_ref):
    x = x_ref[...]
    f32 = jnp.float32
    u1 = jnp.tanh(jnp.dot(w1_ref[...], x, preferred_element_type=f32)
                  + b1_ref[...])
    h1 = jax.nn.relu((1.0 - u1[:32, :]) * u1[32:, :])
    u2 = jnp.tanh(jnp.dot(w2_ref[...], h1, preferred_element_type=f32)
                  + b2_ref[...])
    h2 = jnp.tanh((0.5 - 0.5 * u2[:128, :]) * u2[128:, :])
    o_ref[...] = (jnp.dot(wf_ref[...], h2, preferred_element_type=f32)
                  + bf_ref[...])


def kernel(x, edge_index, W_xz1, b_xz1, W_hz1, b_hz1, W_xr1, b_xr1, W_hr1,
           b_hr1, W_xh1, b_xh1, W_hh1, b_hh1, W_xz2, b_xz2, W_hz2, b_hz2,
           W_xr2, b_xr2, W_hr2, b_hr2, W_xh2, b_xh2, W_hh2, b_hh2,
           W_lin, b_lin, W_grad, b_grad):
    del edge_index
    del W_hz1, W_xr1, b_xr1, W_hr1, b_hr1, W_hh1
    del W_hz2, W_xr2, b_xr2, W_hr2, b_hr2, W_hh2
    w1 = jnp.concatenate([0.5 * W_xz1.T, W_xh1.T], axis=0)
    b1 = jnp.concatenate([0.5 * (b_xz1 + b_hz1),
                          b_xh1 + b_hh1]).reshape(64, 1)
    w2 = jnp.concatenate([0.25 * W_xz2.T, 0.5 * W_xh2.T], axis=0)
    b2 = jnp.concatenate([0.5 * (b_xz2 + b_hz2),
                          b_xh2 + b_hh2]).reshape(256, 1)
    wf = W_lin @ W_grad
    bf = b_lin[0] * W_grad[0] + b_grad
    wf = jnp.concatenate([wf[:, 1:], -wf[:, :1]], axis=1).T
    bf = jnp.stack([bf[1], -bf[0]]).reshape(2, 1)

    xt = x.T
    grid = -(-_N // _TILE)
    col_spec = pl.BlockSpec((2, _TILE), lambda i: (0, i))

    def w_spec(a, b):
        return pl.BlockSpec((a, b), lambda i: (0, 0))

    out_t = pl.pallas_call(
        _body,
        grid=(grid,),
        in_specs=[
            col_spec,
            w_spec(64, 2), w_spec(64, 1), w_spec(256, 32), w_spec(256, 1),
            w_spec(2, 128), w_spec(2, 1),
        ],
        out_specs=col_spec,
        out_shape=jax.ShapeDtypeStruct((2, _N), jnp.float32),
        compiler_params=pltpu.CompilerParams(
            dimension_semantics=("parallel",),
        ),
    )(xt, w1, b1, w2, b2, wf, bf)
    return out_t.T
